# baseline (speedup 1.0000x reference)
import functools

import jax
import jax.numpy as jnp
from jax import lax
from jax.experimental import pallas as pl
from jax.experimental.pallas import tpu as pltpu
from jax.experimental.pallas import tpu_sc as plsc

N = 10000
E = 320000
NC = 2
NS = 16
CHUNK = 80
CCHUNK = 80
STAGE_EDGE = 25
STAGE_COL = 50
BAND = 632
LAST_BAND = N - (NS - 1) * BAND


def _segsum_body(edge_split, n_edges, stage_chunks, depth, *refs):
    (z2, srcH, dstH2, zerosH, out, src_v, dst_v) = refs[:7]
    bufs = refs[7:7 + depth]
    acc = refs[7 + depth]
    sems = refs[8 + depth:8 + 2 * depth]

    n_stage = stage_chunks * CHUNK
    halves = n_edges // n_stage
    nch = stage_chunks
    c = lax.axis_index("c")
    s = lax.axis_index("s")
    widx = c * NS + s if edge_split else s

    def for_band(fn):
        @pl.when(s < NS - 1)
        def _():
            fn(s * BAND, BAND)

        @pl.when(s == NS - 1)
        def _():
            fn((NS - 1) * BAND, LAST_BAND)

    for_band(lambda lo, n: pltpu.sync_copy(zerosH.at[pl.ds(lo, n)],
                                           acc.at[pl.ds(lo, n)]))

    plsc.subcore_barrier()

    ebase = widx * n_edges

    def run_stage(h):
        dbase = ebase + h * n_stage

        def issue(k, b):
            pltpu.async_copy(
                z2.at[src_v.at[pl.ds(k * CHUNK, CHUNK)]], bufs[b], sems[b])

        def wait(k, b):
            pltpu.make_async_copy(
                z2.at[src_v.at[pl.ds(k * CHUNK, CHUNK)]], bufs[b],
                sems[b]).wait()

        def scat(k, b):
            pltpu.sync_copy(bufs[b], acc.at[dst_v.at[k, 0]], add=True)

        pltpu.sync_copy(srcH.at[pl.ds(dbase, n_stage)], src_v)
        if not edge_split:
            def scale_body(i, _):
                v = src_v[pl.ds(i * 16, 16)]
                src_v[pl.ds(i * 16, 16)] = v * 2 + c
                return 0
            lax.fori_loop(0, n_stage // 16, scale_body, 0)
        pltpu.sync_copy(dstH2.at[dbase // n_stage], dst_v)

        for k in range(depth - 1):
            issue(k, k)

        def group_body(q, _):
            k = q * depth
            for j in range(depth):
                wait(k + j, j)
                scat(k + j, j)
                issue(k + j + depth - 1, (j + depth - 1) % depth)
            return 0
        steady = (nch - (2 * depth - 3)) // depth
        lax.fori_loop(0, steady, group_body, 0)

        for k in range(depth * steady, nch):
            wait(k, k % depth)
            scat(k, k % depth)
            if k + depth - 1 < nch:
                issue(k + depth - 1, (k + depth - 1) % depth)

    for h in range(halves):
        run_stage(h)

    plsc.subcore_barrier()

    for_band(lambda lo, n: pltpu.sync_copy(acc.at[pl.ds(lo, n)],
                                           out.at[c, pl.ds(lo, n)]))


@functools.lru_cache(maxsize=None)
def _make_segsum(mode):
    edge_split = mode != "col"
    n_edges = E // (NC * NS) if edge_split else E // NS
    stage_chunks = STAGE_EDGE if edge_split else STAGE_COL
    depth = 4 if edge_split else 3
    n_stage = stage_chunks * CHUNK
    mesh = plsc.VectorSubcoreMesh(core_axis_name="c", subcore_axis_name="s")
    dt = jnp.float32
    row = (128,)
    out_type = [jax.ShapeDtypeStruct((NC, N) + row, dt)]
    scratch = (
        [pltpu.VMEM((n_stage,), jnp.int32),
         pltpu.VMEM((stage_chunks, 1, CHUNK), jnp.int32)]
        + [pltpu.VMEM((CHUNK,) + row, dt) for _ in range(depth)]
        + [pltpu.VMEM_SHARED((N,) + row, dt)]
        + [pltpu.SemaphoreType.DMA for _ in range(depth)]
    )
    return pl.kernel(
        functools.partial(_segsum_body, edge_split, n_edges, stage_chunks,
                          depth),
        out_type=tuple(out_type),
        mesh=mesh,
        scratch_types=tuple(scratch),
    )


def _count_body(*refs):
    (dstH, zerosH, onesH, cnt_out,
     dstb0, dstb1, ones_v, acc, semd0, semd1) = refs
    n_edges = E // (NC * NS)
    nch = n_edges // CCHUNK
    c = lax.axis_index("c")
    s = lax.axis_index("s")
    dbase = (c * NS + s) * n_edges

    def for_band(fn):
        @pl.when(s < NS - 1)
        def _():
            fn(s * BAND, BAND)

        @pl.when(s == NS - 1)
        def _():
            fn((NS - 1) * BAND, LAST_BAND)

    for_band(lambda lo, n: pltpu.sync_copy(zerosH.at[pl.ds(lo, n)],
                                           acc.at[pl.ds(lo, n)]))
    pltpu.sync_copy(onesH, ones_v)
    plsc.subcore_barrier()

    def issue(k, dstb, semd):
        pltpu.async_copy(dstH.at[pl.ds(dbase + k * CCHUNK, CCHUNK)], dstb, semd)

    def wait(k, dstb, semd):
        pltpu.make_async_copy(
            dstH.at[pl.ds(dbase + k * CCHUNK, CCHUNK)], dstb, semd).wait()

    def scat(dstb):
        pltpu.sync_copy(ones_v, acc.at[dstb], add=True)

    issue(0, dstb0, semd0)

    def pair_body(p, _):
        k = p * 2
        issue(k + 1, dstb1, semd1)
        wait(k, dstb0, semd0)
        scat(dstb0)
        issue(k + 2, dstb0, semd0)
        wait(k + 1, dstb1, semd1)
        scat(dstb1)
        return 0
    lax.fori_loop(0, (nch - 1) // 2, pair_body, 0)

    wait(nch - 1, dstb0, semd0)
    scat(dstb0)

    plsc.subcore_barrier()
    for_band(lambda lo, n: pltpu.sync_copy(acc.at[pl.ds(lo, n)],
                                           cnt_out.at[c, pl.ds(lo, n)]))


@functools.lru_cache(maxsize=None)
def _make_count():
    mesh = plsc.VectorSubcoreMesh(core_axis_name="c", subcore_axis_name="s")
    return pl.kernel(
        _count_body,
        out_type=(jax.ShapeDtypeStruct((NC, N, 128), jnp.float32),),
        mesh=mesh,
        scratch_types=(
            pltpu.VMEM((CCHUNK,), jnp.int32),
            pltpu.VMEM((CCHUNK,), jnp.int32),
            pltpu.VMEM((CCHUNK, 128), jnp.float32),
            pltpu.VMEM_SHARED((N, 128), jnp.float32),
            pltpu.SemaphoreType.DMA,
            pltpu.SemaphoreType.DMA,
        ),
    )


def _bdot(a, b):
    return jnp.dot(a.astype(jnp.bfloat16), b.astype(jnp.bfloat16),
                   preferred_element_type=jnp.float32)


def _enc_body(x_ref, w_ref, b_ref, o_ref):
    o_ref[...] = _bdot(x_ref[...], w_ref[...]) + b_ref[...]


def _bn_relu(y, g, be):
    mu = jnp.mean(y, axis=0, keepdims=True)
    var = jnp.mean((y - mu) * (y - mu), axis=0, keepdims=True)
    return jnp.maximum((y - mu) * lax.rsqrt(var + 1e-5) * g + be, 0.0)


def _stage_body(s_ref, cnt_ref, z_ref, wt_ref, wb_ref, wr_ref, bl_ref,
                g_ref, be_ref, o_ref):
    r = 1.0 / jnp.maximum(cnt_ref[...], 1.0)
    y = (_bdot(s_ref[0, :, :] * r, wt_ref[...])
         + _bdot(s_ref[1, :, :] * r, wb_ref[...])
         + _bdot(z_ref[...], wr_ref[...])
         + bl_ref[...])
    o_ref[...] = _bn_relu(y, g_ref[...], be_ref[...])


def _stage_proj_body(s_ref, cnt_ref, z_ref, wt_ref, wb_ref, wr_ref, bl_ref,
                     g_ref, be_ref, b1_ref, o_ref, o2_ref):
    r = 1.0 / jnp.maximum(cnt_ref[...], 1.0)
    y = (_bdot(s_ref[0, :, :] * r, wt_ref[...])
         + _bdot(s_ref[1, :, :] * r, wb_ref[...])
         + _bdot(z_ref[...], wr_ref[...])
         + bl_ref[...])
    o = _bn_relu(y, g_ref[...], be_ref[...])
    o_ref[...] = o
    o2_ref[...] = _bdot(o, b1_ref[...])


def _final_body(s_ref, cnt_ref, z_ref, w2_ref, c_ref, o_ref):
    r = 1.0 / jnp.maximum(cnt_ref[...], 1.0)
    o_ref[...] = ((s_ref[0, :, :] + s_ref[1, :, :]) * r
                  + _bdot(z_ref[...], w2_ref[...])
                  + c_ref[...])


def _tc(body, out_shape, *args):
    return pl.pallas_call(body, out_shape=out_shape)(*args)


def kernel(x, edge_index, enc_W, enc_b, Wl0, bl0, Wr0, Wl1, bl1, Wr1,
           Wl2, bl2, Wr2, Wl3, bl3, Wr3, g0, be0, g1, be1, g2, be2,
           dec_W, dec_b):
    f32 = jnp.float32
    src = edge_index[0]
    dst = edge_index[1]
    dst2e = dst.reshape(E // (STAGE_EDGE * CHUNK), STAGE_EDGE, 1, CHUNK)
    dst2c = dst.reshape(E // (STAGE_COL * CHUNK), STAGE_COL, 1, CHUNK)
    zeros128 = jnp.zeros((N, 128), f32)
    ones128 = jnp.ones((CCHUNK, 128), f32)

    z = _tc(_enc_body, jax.ShapeDtypeStruct((N, 128), f32),
            x, enc_W, enc_b.reshape(1, 128))

    (cnt2,) = _make_count()(dst, zeros128, ones128)
    cnt = cnt2[0, :, :1] + cnt2[1, :, :1]

    (s,) = _make_segsum("edge")(z, src, dst2e, zeros128)
    z = _tc(_stage_body, jax.ShapeDtypeStruct((N, 256), f32),
            s, cnt, z, Wl0, Wl0, Wr0, bl0.reshape(1, 256),
            g0.reshape(1, 256), be0.reshape(1, 256))

    (s,) = _make_segsum("col")(z.reshape(2 * N, 128), src, dst2c, zeros128)
    z = _tc(_stage_body, jax.ShapeDtypeStruct((N, 256), f32),
            s, cnt, z, Wl1[:128], Wl1[128:], Wr1, bl1.reshape(1, 256),
            g1.reshape(1, 256), be1.reshape(1, 256))

    B1 = Wl3 @ dec_W
    B2 = Wr3 @ dec_W
    c = bl3 @ dec_W + dec_b

    (s,) = _make_segsum("col")(z.reshape(2 * N, 128), src, dst2c, zeros128)
    z, zp = _tc(_stage_proj_body,
                (jax.ShapeDtypeStruct((N, 256), f32),
                 jax.ShapeDtypeStruct((N, 128), f32)),
                s, cnt, z, Wl2[:128], Wl2[128:], Wr2, bl2.reshape(1, 256),
                g2.reshape(1, 256), be2.reshape(1, 256), B1)

    (s,) = _make_segsum("edge")(zp, src, dst2e, zeros128)
    z = _tc(_final_body, jax.ShapeDtypeStruct((N, 128), f32),
            s, cnt, z, B2, c.reshape(1, 128))
    return z


# speedup vs baseline: 1.0650x; 1.0650x over previous
import functools

import jax
import jax.numpy as jnp
from jax import lax
from jax.experimental import pallas as pl
from jax.experimental.pallas import tpu as pltpu
from jax.experimental.pallas import tpu_sc as plsc

N = 10000
E = 320000
NC = 2
NS = 16
CHUNK = 80
CCHUNK = 80
STAGE_EDGE = 25
STAGE_COL = 25
BAND = 632
LAST_BAND = N - (NS - 1) * BAND


def _segsum_body(edge_split, n_edges, stage_chunks, depth, *refs):
    (z2, srcH, dstH2, zerosH, out, src_v, dst_v) = refs[:7]
    bufs = refs[7:7 + depth]
    acc = refs[7 + depth]
    sems = refs[8 + depth:8 + 2 * depth]

    n_stage = stage_chunks * CHUNK
    halves = n_edges // n_stage
    nch = stage_chunks
    c = lax.axis_index("c")
    s = lax.axis_index("s")
    widx = c * NS + s if edge_split else s

    def for_band(fn):
        @pl.when(s < NS - 1)
        def _():
            fn(s * BAND, BAND)

        @pl.when(s == NS - 1)
        def _():
            fn((NS - 1) * BAND, LAST_BAND)

    for_band(lambda lo, n: pltpu.sync_copy(zerosH.at[pl.ds(lo, n)],
                                           acc.at[pl.ds(lo, n)]))

    plsc.subcore_barrier()

    ebase = widx * n_edges

    def run_stage(h):
        dbase = ebase + h * n_stage

        def issue(k, b):
            pltpu.async_copy(
                z2.at[src_v.at[pl.ds(k * CHUNK, CHUNK)]], bufs[b], sems[b])

        def wait(k, b):
            pltpu.make_async_copy(
                z2.at[src_v.at[pl.ds(k * CHUNK, CHUNK)]], bufs[b],
                sems[b]).wait()

        def scat(k, b):
            pltpu.sync_copy(bufs[b], acc.at[dst_v.at[k, 0]], add=True)

        pltpu.sync_copy(srcH.at[pl.ds(dbase, n_stage)], src_v)
        if not edge_split:
            def scale_body(i, _):
                v = src_v[pl.ds(i * 16, 16)]
                src_v[pl.ds(i * 16, 16)] = v * 2 + c
                return 0
            lax.fori_loop(0, n_stage // 16, scale_body, 0)
        pltpu.sync_copy(dstH2.at[dbase // n_stage], dst_v)

        for k in range(depth - 1):
            issue(k, k)

        def group_body(q, _):
            k = q * depth
            for j in range(depth):
                wait(k + j, j)
                scat(k + j, j)
                issue(k + j + depth - 1, (j + depth - 1) % depth)
            return 0
        steady = (nch - (2 * depth - 3)) // depth
        lax.fori_loop(0, steady, group_body, 0)

        for k in range(depth * steady, nch):
            wait(k, k % depth)
            scat(k, k % depth)
            if k + depth - 1 < nch:
                issue(k + depth - 1, (k + depth - 1) % depth)

    for h in range(halves):
        run_stage(h)

    plsc.subcore_barrier()

    for_band(lambda lo, n: pltpu.sync_copy(acc.at[pl.ds(lo, n)],
                                           out.at[c, pl.ds(lo, n)]))


@functools.lru_cache(maxsize=None)
def _make_segsum(mode):
    edge_split = mode != "col"
    n_edges = E // (NC * NS) if edge_split else E // NS
    stage_chunks = STAGE_EDGE if edge_split else STAGE_COL
    depth = 4
    n_stage = stage_chunks * CHUNK
    mesh = plsc.VectorSubcoreMesh(core_axis_name="c", subcore_axis_name="s")
    dt = jnp.float32
    row = (128,)
    out_type = [jax.ShapeDtypeStruct((NC, N) + row, dt)]
    scratch = (
        [pltpu.VMEM((n_stage,), jnp.int32),
         pltpu.VMEM((stage_chunks, 1, CHUNK), jnp.int32)]
        + [pltpu.VMEM((CHUNK,) + row, dt) for _ in range(depth)]
        + [pltpu.VMEM_SHARED((N,) + row, dt)]
        + [pltpu.SemaphoreType.DMA for _ in range(depth)]
    )
    return pl.kernel(
        functools.partial(_segsum_body, edge_split, n_edges, stage_chunks,
                          depth),
        out_type=tuple(out_type),
        mesh=mesh,
        scratch_types=tuple(scratch),
    )


def _count_body(*refs):
    (dstH, zerosH, onesH, cnt_out,
     dstb0, dstb1, ones_v, acc, semd0, semd1) = refs
    n_edges = E // (NC * NS)
    nch = n_edges // CCHUNK
    c = lax.axis_index("c")
    s = lax.axis_index("s")
    dbase = (c * NS + s) * n_edges

    def for_band(fn):
        @pl.when(s < NS - 1)
        def _():
            fn(s * BAND, BAND)

        @pl.when(s == NS - 1)
        def _():
            fn((NS - 1) * BAND, LAST_BAND)

    for_band(lambda lo, n: pltpu.sync_copy(zerosH.at[pl.ds(lo, n)],
                                           acc.at[pl.ds(lo, n)]))
    pltpu.sync_copy(onesH, ones_v)
    plsc.subcore_barrier()

    def issue(k, dstb, semd):
        pltpu.async_copy(dstH.at[pl.ds(dbase + k * CCHUNK, CCHUNK)], dstb, semd)

    def wait(k, dstb, semd):
        pltpu.make_async_copy(
            dstH.at[pl.ds(dbase + k * CCHUNK, CCHUNK)], dstb, semd).wait()

    def scat(dstb):
        pltpu.sync_copy(ones_v, acc.at[dstb], add=True)

    issue(0, dstb0, semd0)

    def pair_body(p, _):
        k = p * 2
        issue(k + 1, dstb1, semd1)
        wait(k, dstb0, semd0)
        scat(dstb0)
        issue(k + 2, dstb0, semd0)
        wait(k + 1, dstb1, semd1)
        scat(dstb1)
        return 0
    lax.fori_loop(0, (nch - 1) // 2, pair_body, 0)

    wait(nch - 1, dstb0, semd0)
    scat(dstb0)

    plsc.subcore_barrier()
    for_band(lambda lo, n: pltpu.sync_copy(acc.at[pl.ds(lo, n)],
                                           cnt_out.at[c, pl.ds(lo, n)]))


@functools.lru_cache(maxsize=None)
def _make_count():
    mesh = plsc.VectorSubcoreMesh(core_axis_name="c", subcore_axis_name="s")
    return pl.kernel(
        _count_body,
        out_type=(jax.ShapeDtypeStruct((NC, N, 128), jnp.float32),),
        mesh=mesh,
        scratch_types=(
            pltpu.VMEM((CCHUNK,), jnp.int32),
            pltpu.VMEM((CCHUNK,), jnp.int32),
            pltpu.VMEM((CCHUNK, 128), jnp.float32),
            pltpu.VMEM_SHARED((N, 128), jnp.float32),
            pltpu.SemaphoreType.DMA,
            pltpu.SemaphoreType.DMA,
        ),
    )


def _bdot(a, b):
    return jnp.dot(a.astype(jnp.bfloat16), b.astype(jnp.bfloat16),
                   preferred_element_type=jnp.float32)


def _enc_body(x_ref, w_ref, b_ref, o_ref):
    o_ref[...] = _bdot(x_ref[...], w_ref[...]) + b_ref[...]


def _bn_relu(y, g, be):
    mu = jnp.mean(y, axis=0, keepdims=True)
    var = jnp.mean((y - mu) * (y - mu), axis=0, keepdims=True)
    return jnp.maximum((y - mu) * lax.rsqrt(var + 1e-5) * g + be, 0.0)


def _stage_body(s_ref, cnt_ref, z_ref, wt_ref, wb_ref, wr_ref, bl_ref,
                g_ref, be_ref, o_ref):
    r = 1.0 / jnp.maximum(cnt_ref[...], 1.0)
    y = (_bdot(s_ref[0, :, :] * r, wt_ref[...])
         + _bdot(s_ref[1, :, :] * r, wb_ref[...])
         + _bdot(z_ref[...], wr_ref[...])
         + bl_ref[...])
    o_ref[...] = _bn_relu(y, g_ref[...], be_ref[...])


def _stage_proj_body(s_ref, cnt_ref, z_ref, wt_ref, wb_ref, wr_ref, bl_ref,
                     g_ref, be_ref, b1_ref, o_ref, o2_ref):
    r = 1.0 / jnp.maximum(cnt_ref[...], 1.0)
    y = (_bdot(s_ref[0, :, :] * r, wt_ref[...])
         + _bdot(s_ref[1, :, :] * r, wb_ref[...])
         + _bdot(z_ref[...], wr_ref[...])
         + bl_ref[...])
    o = _bn_relu(y, g_ref[...], be_ref[...])
    o_ref[...] = o
    o2_ref[...] = _bdot(o, b1_ref[...])


def _final_body(s_ref, cnt_ref, z_ref, w2_ref, c_ref, o_ref):
    r = 1.0 / jnp.maximum(cnt_ref[...], 1.0)
    o_ref[...] = ((s_ref[0, :, :] + s_ref[1, :, :]) * r
                  + _bdot(z_ref[...], w2_ref[...])
                  + c_ref[...])


def _tc(body, out_shape, *args):
    return pl.pallas_call(body, out_shape=out_shape)(*args)


def kernel(x, edge_index, enc_W, enc_b, Wl0, bl0, Wr0, Wl1, bl1, Wr1,
           Wl2, bl2, Wr2, Wl3, bl3, Wr3, g0, be0, g1, be1, g2, be2,
           dec_W, dec_b):
    f32 = jnp.float32
    src = edge_index[0]
    dst = edge_index[1]
    dst2e = dst.reshape(E // (STAGE_EDGE * CHUNK), STAGE_EDGE, 1, CHUNK)
    dst2c = dst.reshape(E // (STAGE_COL * CHUNK), STAGE_COL, 1, CHUNK)
    zeros128 = jnp.zeros((N, 128), f32)
    ones128 = jnp.ones((CCHUNK, 128), f32)

    z = _tc(_enc_body, jax.ShapeDtypeStruct((N, 128), f32),
            x, enc_W, enc_b.reshape(1, 128))

    (cnt2,) = _make_count()(dst, zeros128, ones128)
    cnt = cnt2[0, :, :1] + cnt2[1, :, :1]

    (s,) = _make_segsum("edge")(z, src, dst2e, zeros128)
    z = _tc(_stage_body, jax.ShapeDtypeStruct((N, 256), f32),
            s, cnt, z, Wl0, Wl0, Wr0, bl0.reshape(1, 256),
            g0.reshape(1, 256), be0.reshape(1, 256))

    (s,) = _make_segsum("col")(z.reshape(2 * N, 128), src, dst2c, zeros128)
    z = _tc(_stage_body, jax.ShapeDtypeStruct((N, 256), f32),
            s, cnt, z, Wl1[:128], Wl1[128:], Wr1, bl1.reshape(1, 256),
            g1.reshape(1, 256), be1.reshape(1, 256))

    B1 = Wl3 @ dec_W
    B2 = Wr3 @ dec_W
    c = bl3 @ dec_W + dec_b

    (s,) = _make_segsum("col")(z.reshape(2 * N, 128), src, dst2c, zeros128)
    z, zp = _tc(_stage_proj_body,
                (jax.ShapeDtypeStruct((N, 256), f32),
                 jax.ShapeDtypeStruct((N, 128), f32)),
                s, cnt, z, Wl2[:128], Wl2[128:], Wr2, bl2.reshape(1, 256),
                g2.reshape(1, 256), be2.reshape(1, 256), B1)

    (s,) = _make_segsum("edge")(zp, src, dst2e, zeros128)
    z = _tc(_final_body, jax.ShapeDtypeStruct((N, 128), f32),
            s, cnt, z, B2, c.reshape(1, 128))
    return z
